# Initial kernel scaffold; baseline (speedup 1.0000x reference)
import jax
import jax.numpy as jnp
from jax import lax
from jax.experimental import pallas as pl
from jax.experimental.pallas import tpu as pltpu
from jax.experimental.pallas import tpu_sc as plsc

N = 10000
E = 320000
D = 128
ED = 16
NP = 10240
EP = 323584
C = 128
NW = 32
EPW = EP // NW
CHUNKS = EPW // C
DH = D + 16
RPT = NP // 16


def _ab_body(x_ref, ws_ref, wd_ref, a_ref, b_ref):
    x = x_ref[...]
    a_ref[...] = jnp.dot(x, ws_ref[...], preferred_element_type=jnp.float32)
    b_ref[...] = jnp.dot(x, wd_ref[...], preferred_element_type=jnp.float32)


def _ce_body(e_ref, we_ref, b_ref, c_ref):
    c_ref[...] = (
        jnp.dot(e_ref[...], we_ref[...], preferred_element_type=jnp.float32)
        + b_ref[...]
    )


def _sc_body(a_hbm, b_hbm, ce_hbm, sidx_hbm, didx_hbm, parts_hbm,
             agg, sidx_v, didx_v, a_v, b_v, c_v, h_v, sem_a, sem_b):
    cid = lax.axis_index("c")
    sid = lax.axis_index("s")
    wid = sid * 2 + cid

    def zrow(i, carry):
        for j in range(DH // 16):
            h_v[i, pl.ds(16 * j, 16)] = jnp.zeros((16,), jnp.float32)
        return carry

    lax.fori_loop(0, C, zrow, 0)
    for r in range(RPT // C):
        pltpu.sync_copy(h_v, agg.at[pl.ds(sid * RPT + r * C, C)])

    one0 = jnp.where(
        lax.iota(jnp.int32, 16) == 0,
        jnp.full((16,), 1.0, jnp.float32),
        jnp.zeros((16,), jnp.float32),
    )

    def orow(i, carry):
        h_v[i, pl.ds(D, 16)] = one0
        return carry

    lax.fori_loop(0, C, orow, 0)
    plsc.subcore_barrier()

    def chunk(g, carry):
        base = wid * EPW + g * C
        pltpu.sync_copy(sidx_hbm.at[pl.ds(base, C)], sidx_v)
        pltpu.sync_copy(didx_hbm.at[pl.ds(base, C)], didx_v)
        cp_a = pltpu.async_copy(a_hbm.at[sidx_v], a_v, sem_a)
        cp_b = pltpu.async_copy(b_hbm.at[didx_v], b_v, sem_b)
        pltpu.sync_copy(ce_hbm.at[pl.ds(base, C)], c_v)
        cp_a.wait()
        cp_b.wait()

        def row(i, rcarry):
            for j in range(D // 16):
                s = pl.ds(16 * j, 16)
                h_v[i, s] = jnp.maximum(a_v[i, s] + b_v[i, s] + c_v[i, s], 0.0)
            return rcarry

        lax.fori_loop(0, C, row, 0)
        pltpu.sync_copy(h_v, agg.at[didx_v], add=True)
        return carry

    lax.fori_loop(0, CHUNKS, chunk, 0)
    plsc.subcore_barrier()
    pltpu.sync_copy(
        agg.at[pl.ds(sid * RPT, RPT)],
        parts_hbm.at[cid, pl.ds(sid * RPT, RPT)],
    )


def _post_body(x_ref, p0_ref, p1_ref, wm2_ref, bm2_ref, wu1x_ref, wu1a_ref,
               bu1_ref, wu2_ref, bu2_ref, g_ref, bt_ref, o_ref):
    p0 = p0_ref[...]
    p1 = p1_ref[...]
    aggh = p0[:, :D] + p1[:, :D]
    deg = p0[:, D:D + 1] + p1[:, D:D + 1]
    x = x_ref[...]
    aggregated = (
        jnp.dot(aggh, wm2_ref[...], preferred_element_type=jnp.float32)
        + deg * bm2_ref[...]
    )
    h2 = jnp.maximum(
        jnp.dot(x, wu1x_ref[...], preferred_element_type=jnp.float32)
        + jnp.dot(aggregated, wu1a_ref[...], preferred_element_type=jnp.float32)
        + bu1_ref[...],
        0.0,
    )
    y = x + jnp.dot(h2, wu2_ref[...], preferred_element_type=jnp.float32) + bu2_ref[...]
    mu = jnp.mean(y, axis=1, keepdims=True)
    var = jnp.mean((y - mu) ** 2, axis=1, keepdims=True)
    o_ref[...] = (y - mu) * lax.rsqrt(var + 1e-5) * g_ref[...] + bt_ref[...]


def kernel(node_features, edge_index, edge_features, Wm1, bm1, Wm2, bm2,
           Wu1, bu1, Wu2, bu2, gamma, beta):
    f32 = jnp.float32
    x_pad = jnp.pad(node_features, ((0, NP - N), (0, 0)))
    e_pad = jnp.pad(edge_features, ((0, EP - E), (0, 0)))
    pad_idx = jnp.full((EP - E,), NP - 1, jnp.int32)
    sidx = jnp.concatenate([edge_index[0].astype(jnp.int32), pad_idx])
    didx = jnp.concatenate([edge_index[1].astype(jnp.int32), pad_idx])

    ws_t = Wm1[:, :D].T
    wd_t = Wm1[:, D:2 * D].T
    we_t = Wm1[:, 2 * D:].T
    bm1_2 = bm1.reshape(1, D)

    a_tab, b_tab = pl.pallas_call(
        _ab_body,
        grid=(NP // 1280,),
        in_specs=[
            pl.BlockSpec((1280, D), lambda i: (i, 0)),
            pl.BlockSpec((D, D), lambda i: (0, 0)),
            pl.BlockSpec((D, D), lambda i: (0, 0)),
        ],
        out_specs=[
            pl.BlockSpec((1280, D), lambda i: (i, 0)),
            pl.BlockSpec((1280, D), lambda i: (i, 0)),
        ],
        out_shape=[
            jax.ShapeDtypeStruct((NP, D), f32),
            jax.ShapeDtypeStruct((NP, D), f32),
        ],
    )(x_pad, ws_t, wd_t)

    EB = 4096
    ce_tab = pl.pallas_call(
        _ce_body,
        grid=(EP // EB,),
        in_specs=[
            pl.BlockSpec((EB, ED), lambda i: (i, 0)),
            pl.BlockSpec((ED, D), lambda i: (0, 0)),
            pl.BlockSpec((1, D), lambda i: (0, 0)),
        ],
        out_specs=pl.BlockSpec((EB, D), lambda i: (i, 0)),
        out_shape=jax.ShapeDtypeStruct((EP, D), f32),
    )(e_pad, we_t, bm1_2)

    parts = pl.kernel(
        _sc_body,
        out_type=jax.ShapeDtypeStruct((2, NP, DH), f32),
        mesh=plsc.VectorSubcoreMesh(core_axis_name="c", subcore_axis_name="s"),
        scratch_types=[
            pltpu.VMEM_SHARED((NP, DH), f32),
            pltpu.VMEM((C,), jnp.int32),
            pltpu.VMEM((C,), jnp.int32),
            pltpu.VMEM((C, D), f32),
            pltpu.VMEM((C, D), f32),
            pltpu.VMEM((C, D), f32),
            pltpu.VMEM((C, DH), f32),
            pltpu.SemaphoreType.DMA,
            pltpu.SemaphoreType.DMA,
        ],
    )(a_tab, b_tab, ce_tab, sidx, didx)

    RB = 1000
    out = pl.pallas_call(
        _post_body,
        grid=(N // RB,),
        in_specs=[
            pl.BlockSpec((RB, D), lambda i: (i, 0)),
            pl.BlockSpec((RB, DH), lambda i: (i, 0)),
            pl.BlockSpec((RB, DH), lambda i: (i, 0)),
            pl.BlockSpec((D, D), lambda i: (0, 0)),
            pl.BlockSpec((1, D), lambda i: (0, 0)),
            pl.BlockSpec((D, D), lambda i: (0, 0)),
            pl.BlockSpec((D, D), lambda i: (0, 0)),
            pl.BlockSpec((1, D), lambda i: (0, 0)),
            pl.BlockSpec((D, D), lambda i: (0, 0)),
            pl.BlockSpec((1, D), lambda i: (0, 0)),
            pl.BlockSpec((1, D), lambda i: (0, 0)),
            pl.BlockSpec((1, D), lambda i: (0, 0)),
        ],
        out_specs=pl.BlockSpec((RB, D), lambda i: (i, 0)),
        out_shape=jax.ShapeDtypeStruct((N, D), f32),
    )(node_features, parts[0], parts[1], Wm2.T, bm2.reshape(1, D),
      Wu1[:, :D].T, Wu1[:, D:].T, bu1.reshape(1, D), Wu2.T,
      bu2.reshape(1, D), gamma.reshape(1, D), beta.reshape(1, D))
    return out


# speedup vs baseline: 2.9232x; 2.9232x over previous
import jax
import jax.numpy as jnp
from jax import lax
from jax.experimental import pallas as pl
from jax.experimental.pallas import tpu as pltpu
from jax.experimental.pallas import tpu_sc as plsc

N = 10000
E = 320000
D = 128
ED = 16
NP = 10240
EP = 321536
C = 64
NW = 32
EPW = EP // NW
CHUNKS = EPW // C
RPT = NP // 16


def _ab_body(x_ref, ws_ref, wd_ref, a_ref, b_ref):
    x = x_ref[...]
    a_ref[...] = jnp.dot(x, ws_ref[...], preferred_element_type=jnp.float32)
    b_ref[...] = jnp.dot(x, wd_ref[...], preferred_element_type=jnp.float32)


def _ce_body(e_ref, we_ref, b_ref, c_ref):
    c_ref[...] = (
        jnp.dot(e_ref[...], we_ref[...], preferred_element_type=jnp.float32)
        + b_ref[...]
    )


def _sc_body(a_hbm, b_hbm, ce_hbm, sidx_hbm, didx_hbm, parts_hbm, degp_hbm,
             agg, sidx_v, didx_v, a_v, b_v, c_v, h_v, degl, sem_a, sem_b):
    cid = lax.axis_index("c")
    sid = lax.axis_index("s")
    wid = sid * 2 + cid

    def zrow(i, carry):
        for j in range(D // 16):
            h_v[i, pl.ds(16 * j, 16)] = jnp.zeros((16,), jnp.float32)
        return carry

    lax.fori_loop(0, C, zrow, 0)

    def zdeg(i, carry):
        degl[pl.ds(16 * i, 16)] = jnp.zeros((16,), jnp.float32)
        return carry

    lax.fori_loop(0, NP // 16, zdeg, 0)
    for r in range(RPT // C):
        pltpu.sync_copy(h_v, agg.at[pl.ds(sid * RPT + r * C, C)])
    plsc.subcore_barrier()

    ones16 = jnp.full((16,), 1.0, jnp.float32)

    def chunk(g, carry):
        base = wid * EPW + g * C
        pltpu.sync_copy(sidx_hbm.at[pl.ds(base, C)], sidx_v)
        pltpu.sync_copy(didx_hbm.at[pl.ds(base, C)], didx_v)
        cp_a = pltpu.async_copy(a_hbm.at[sidx_v], a_v, sem_a)
        cp_b = pltpu.async_copy(b_hbm.at[didx_v], b_v, sem_b)
        pltpu.sync_copy(ce_hbm.at[pl.ds(base, C)], c_v)
        for k in range(C // 16):
            idx16 = didx_v[pl.ds(16 * k, 16)]
            plsc.addupdate_scatter(degl, [idx16], ones16)
        cp_a.wait()
        cp_b.wait()

        def row(i, rcarry):
            for j in range(D // 16):
                s = pl.ds(16 * j, 16)
                h_v[i, s] = jnp.maximum(a_v[i, s] + b_v[i, s] + c_v[i, s], 0.0)
            return rcarry

        lax.fori_loop(0, C, row, 0)
        pltpu.sync_copy(h_v, agg.at[didx_v], add=True)
        return carry

    lax.fori_loop(0, CHUNKS, chunk, 0)
    pltpu.sync_copy(degl, degp_hbm.at[wid])
    plsc.subcore_barrier()
    pltpu.sync_copy(
        agg.at[pl.ds(sid * RPT, RPT)],
        parts_hbm.at[cid, pl.ds(sid * RPT, RPT)],
    )


def _deg_body(ones_ref, d_ref, o_ref):
    o_ref[...] = jnp.dot(ones_ref[...], d_ref[...],
                         preferred_element_type=jnp.float32)


def _post_body(x_ref, p0_ref, p1_ref, deg_ref, wm2_ref, bm2_ref,
               wu1x_ref, wu1a_ref, bu1_ref, wu2_ref, bu2_ref, g_ref, bt_ref,
               o_ref):
    aggh = p0_ref[...] + p1_ref[...]
    deg = deg_ref[...]
    x = x_ref[...]
    aggregated = (
        jnp.dot(aggh, wm2_ref[...], preferred_element_type=jnp.float32)
        + deg * bm2_ref[...]
    )
    h2 = jnp.maximum(
        jnp.dot(x, wu1x_ref[...], preferred_element_type=jnp.float32)
        + jnp.dot(aggregated, wu1a_ref[...], preferred_element_type=jnp.float32)
        + bu1_ref[...],
        0.0,
    )
    y = x + jnp.dot(h2, wu2_ref[...], preferred_element_type=jnp.float32) + bu2_ref[...]
    mu = jnp.mean(y, axis=1, keepdims=True)
    var = jnp.mean((y - mu) ** 2, axis=1, keepdims=True)
    o_ref[...] = (y - mu) * lax.rsqrt(var + 1e-5) * g_ref[...] + bt_ref[...]


def kernel(node_features, edge_index, edge_features, Wm1, bm1, Wm2, bm2,
           Wu1, bu1, Wu2, bu2, gamma, beta):
    f32 = jnp.float32
    x_pad = jnp.pad(node_features, ((0, NP - N), (0, 0)))
    e_pad = jnp.pad(edge_features, ((0, EP - E), (0, 0)))
    pad_idx = jnp.full((EP - E,), NP - 1, jnp.int32)
    sidx = jnp.concatenate([edge_index[0].astype(jnp.int32), pad_idx])
    didx = jnp.concatenate([edge_index[1].astype(jnp.int32), pad_idx])

    ws_t = Wm1[:, :D].T
    wd_t = Wm1[:, D:2 * D].T
    we_t = Wm1[:, 2 * D:].T
    bm1_2 = bm1.reshape(1, D)

    a_tab, b_tab = pl.pallas_call(
        _ab_body,
        grid=(NP // 1280,),
        in_specs=[
            pl.BlockSpec((1280, D), lambda i: (i, 0)),
            pl.BlockSpec((D, D), lambda i: (0, 0)),
            pl.BlockSpec((D, D), lambda i: (0, 0)),
        ],
        out_specs=[
            pl.BlockSpec((1280, D), lambda i: (i, 0)),
            pl.BlockSpec((1280, D), lambda i: (i, 0)),
        ],
        out_shape=[
            jax.ShapeDtypeStruct((NP, D), f32),
            jax.ShapeDtypeStruct((NP, D), f32),
        ],
    )(x_pad, ws_t, wd_t)

    EB = 2048
    ce_tab = pl.pallas_call(
        _ce_body,
        grid=(EP // EB,),
        in_specs=[
            pl.BlockSpec((EB, ED), lambda i: (i, 0)),
            pl.BlockSpec((ED, D), lambda i: (0, 0)),
            pl.BlockSpec((1, D), lambda i: (0, 0)),
        ],
        out_specs=pl.BlockSpec((EB, D), lambda i: (i, 0)),
        out_shape=jax.ShapeDtypeStruct((EP, D), f32),
    )(e_pad, we_t, bm1_2)

    parts, degp = pl.kernel(
        _sc_body,
        out_type=(
            jax.ShapeDtypeStruct((2, NP, D), f32),
            jax.ShapeDtypeStruct((NW, NP), f32),
        ),
        mesh=plsc.VectorSubcoreMesh(core_axis_name="c", subcore_axis_name="s"),
        compiler_params=pltpu.CompilerParams(needs_layout_passes=False),
        scratch_types=[
            pltpu.VMEM_SHARED((NP, D), f32),
            pltpu.VMEM((C,), jnp.int32),
            pltpu.VMEM((C,), jnp.int32),
            pltpu.VMEM((C, D), f32),
            pltpu.VMEM((C, D), f32),
            pltpu.VMEM((C, D), f32),
            pltpu.VMEM((C, D), f32),
            pltpu.VMEM((NP,), f32),
            pltpu.SemaphoreType.DMA,
            pltpu.SemaphoreType.DMA,
        ],
    )(a_tab, b_tab, ce_tab, sidx, didx)

    degsum = pl.pallas_call(
        _deg_body,
        grid=(1,),
        in_specs=[
            pl.BlockSpec((1, NW), lambda i: (0, 0)),
            pl.BlockSpec((NW, NP), lambda i: (0, 0)),
        ],
        out_specs=pl.BlockSpec((1, NP), lambda i: (0, 0)),
        out_shape=jax.ShapeDtypeStruct((1, NP), f32),
    )(jnp.ones((1, NW), f32), degp)
    deg_col = degsum.reshape(NP, 1)[:N]

    RB = 1000
    out = pl.pallas_call(
        _post_body,
        grid=(N // RB,),
        in_specs=[
            pl.BlockSpec((RB, D), lambda i: (i, 0)),
            pl.BlockSpec((RB, D), lambda i: (i, 0)),
            pl.BlockSpec((RB, D), lambda i: (i, 0)),
            pl.BlockSpec((RB, 1), lambda i: (i, 0)),
            pl.BlockSpec((D, D), lambda i: (0, 0)),
            pl.BlockSpec((1, D), lambda i: (0, 0)),
            pl.BlockSpec((D, D), lambda i: (0, 0)),
            pl.BlockSpec((D, D), lambda i: (0, 0)),
            pl.BlockSpec((1, D), lambda i: (0, 0)),
            pl.BlockSpec((D, D), lambda i: (0, 0)),
            pl.BlockSpec((1, D), lambda i: (0, 0)),
            pl.BlockSpec((1, D), lambda i: (0, 0)),
            pl.BlockSpec((1, D), lambda i: (0, 0)),
        ],
        out_specs=pl.BlockSpec((RB, D), lambda i: (i, 0)),
        out_shape=jax.ShapeDtypeStruct((N, D), f32),
    )(node_features, parts[0], parts[1], deg_col, Wm2.T, bm2.reshape(1, D),
      Wu1[:, :D].T, Wu1[:, D:].T, bu1.reshape(1, D), Wu2.T,
      bu2.reshape(1, D), gamma.reshape(1, D), beta.reshape(1, D))
    return out
